# baseline (speedup 1.0000x reference)
import jax
import jax.numpy as jnp
from jax import lax
from jax.experimental import pallas as pl
from jax.experimental.pallas import tpu as pltpu
from jax.experimental.pallas import tpu_sc as plsc

N = 10000
F = 128
NB = 80
NSLOT = 4
TILES = 16
CHUNKS_PER_CORE = 4
ROWS_PER_TILE = 624


def _hop_body(hsrc, epk, out, acc, gbuf, egb, rowb, idxb,
              sem_e, sem_g, sem_s):
    c = lax.axis_index("c")
    s = lax.axis_index("s")
    nbatch = epk.shape[0] // (TILES * 3 * NB)
    ngroup = nbatch // NSLOT
    r0 = s * ROWS_PER_TILE
    tail = N - TILES * ROWS_PER_TILE

    def chunk_body(k, _unused):
        g = c * CHUNKS_PER_CORE + k
        base = g * N

        def zrow(i, _):
            for j in range(8):
                gbuf[0, i, pl.ds(j * 16, 16)] = jnp.zeros((16,), jnp.float32)
            return 0
        lax.fori_loop(0, NB, zrow, 0)
        for t in range(ROWS_PER_TILE // NB):
            pltpu.sync_copy(gbuf.at[0], acc.at[pl.ds(r0 + t * NB, NB)])
        rem = ROWS_PER_TILE % NB
        if rem:
            pltpu.sync_copy(gbuf.at[0, pl.ds(0, rem)],
                            acc.at[pl.ds(r0 + (ROWS_PER_TILE // NB) * NB, rem)])

        @pl.when(s == TILES - 1)
        def _():
            pltpu.sync_copy(gbuf.at[0, pl.ds(0, tail)],
                            acc.at[pl.ds(TILES * ROWS_PER_TILE, tail)])
        plsc.subcore_barrier()

        bslot0 = s * nbatch

        def E(m, q):
            pltpu.async_copy(epk.at[pl.ds((bslot0 + m) * 3 * NB, 3 * NB)],
                             egb.at[pl.ds(q * 3 * NB, 3 * NB)], sem_e.at[q])

        def WE(q):
            pltpu.make_async_copy(epk.at[pl.ds(0, 3 * NB)],
                                  egb.at[pl.ds(q * 3 * NB, 3 * NB)],
                                  sem_e.at[q]).wait()

        def X(q):
            e0 = q * 3 * NB
            for j in range(NB // 16):
                idxb[q, pl.ds(j * 16, 16)] = egb[pl.ds(e0 + j * 16, 16)] + base
            for j in range(NB // 16):
                rowb[q, pl.ds(j * 16, 16)] = egb[pl.ds(e0 + 2 * NB + j * 16, 16)]

        def G(q):
            pltpu.async_copy(hsrc.at[idxb.at[q]], gbuf.at[q], sem_g.at[q])

        def WG(q):
            pltpu.make_async_copy(hsrc.at[pl.ds(0, NB)], gbuf.at[q], sem_g.at[q]).wait()

        def S(q):
            def scale(eb, _):
                vv = egb[pl.ds(q * 3 * NB + NB + eb * 16, 16)].astype(jnp.float32) * (2.0 ** -30)
                for e2 in range(16):
                    v = vv[e2]
                    e = eb * 16 + e2
                    for j in range(8):
                        gbuf[q, e, pl.ds(j * 16, 16)] = gbuf[q, e, pl.ds(j * 16, 16)] * v
                return 0
            lax.fori_loop(0, NB // 16, scale, 0)

        def C(q):
            pltpu.async_copy(gbuf.at[q], acc.at[rowb.at[q]], sem_s.at[q], add=True)

        def WS(q):
            pltpu.make_async_copy(hsrc.at[pl.ds(0, NB)], gbuf.at[q], sem_s.at[q]).wait()

        E(0, 0)
        E(1, 1)
        WE(0)
        X(0)
        G(0)

        def group(gi, _):
            m0 = gi * NSLOT
            for off in range(NSLOT):
                m = m0 + off

                @pl.when(m >= 2)
                def _():
                    WS((off + 2) % NSLOT)

                @pl.when(m + 2 <= nbatch - 1)
                def _():
                    E(m + 2, (off + 2) % NSLOT)

                @pl.when(m + 1 <= nbatch - 1)
                def _():
                    WE((off + 1) % NSLOT)
                    X((off + 1) % NSLOT)
                    G((off + 1) % NSLOT)
                WG(off)
                S(off)
                C(off)
            return 0
        lax.fori_loop(0, ngroup, group, 0)

        WS(2)
        WS(3)

        plsc.subcore_barrier()

        pltpu.sync_copy(acc.at[pl.ds(r0, ROWS_PER_TILE)],
                        out.at[pl.ds(g * N + r0, ROWS_PER_TILE)])

        @pl.when(s == TILES - 1)
        def _():
            pltpu.sync_copy(acc.at[pl.ds(TILES * ROWS_PER_TILE, tail)],
                            out.at[pl.ds(g * N + TILES * ROWS_PER_TILE, tail)])
        plsc.subcore_barrier()
        return 0

    lax.fori_loop(0, CHUNKS_PER_CORE, chunk_body, 0)


def _linear_body(h_ref, w_ref, b_ref, o_ref):
    o_ref[...] = jnp.dot(h_ref[...], w_ref[...],
                         preferred_element_type=jnp.float32) + b_ref[...]


def kernel(x, edge_row, edge_col, edge_val, W, b):
    B_, N_, F_ = x.shape
    E = edge_row.shape[0]

    unit = TILES * NB * NSLOT
    EP = ((E + unit - 1) // unit) * unit
    pad = EP - E
    ar = (jnp.arange(pad, dtype=jnp.int32) % N_)
    rowp = jnp.concatenate([edge_row, ar])
    colp = jnp.concatenate([edge_col, ar])
    valp = jnp.concatenate([edge_val, jnp.zeros((pad,), jnp.float32)])
    vbits = jnp.round(valp * (2.0 ** 30)).astype(jnp.int32)
    epk = jnp.stack([colp.reshape(EP // NB, NB),
                     vbits.reshape(EP // NB, NB),
                     rowp.reshape(EP // NB, NB)], axis=1).reshape(-1)

    mesh = plsc.VectorSubcoreMesh(core_axis_name="c", subcore_axis_name="s")
    hop = pl.kernel(
        _hop_body,
        mesh=mesh,
        out_type=jax.ShapeDtypeStruct((B_ * N_, F_), jnp.float32),
        scratch_types=[
            pltpu.VMEM_SHARED((N_, F_), jnp.float32),
            pltpu.VMEM((NSLOT, NB, F_), jnp.float32),
            pltpu.VMEM((NSLOT * 3 * NB,), jnp.int32),
            pltpu.VMEM((NSLOT, NB), jnp.int32),
            pltpu.VMEM((NSLOT, NB), jnp.int32),
            pltpu.SemaphoreType.DMA((NSLOT,)),
            pltpu.SemaphoreType.DMA((NSLOT,)),
            pltpu.SemaphoreType.DMA((NSLOT,)),
        ],
    )

    h = x.reshape(B_ * N_, F_)
    h = hop(h, epk)
    h = hop(h, epk)

    M = B_ * N_
    BLK = 2000
    out = pl.pallas_call(
        _linear_body,
        grid=(M // BLK,),
        in_specs=[
            pl.BlockSpec((BLK, F_), lambda i: (i, 0)),
            pl.BlockSpec((F_, W.shape[1]), lambda i: (0, 0)),
            pl.BlockSpec((W.shape[1],), lambda i: (0,)),
        ],
        out_specs=pl.BlockSpec((BLK, W.shape[1]), lambda i: (i, 0)),
        out_shape=jax.ShapeDtypeStruct((M, W.shape[1]), jnp.float32),
    )(h, W, b)
    return out.reshape(B_, N_, W.shape[1])


# speedup vs baseline: 1.0575x; 1.0575x over previous
import jax
import jax.numpy as jnp
from jax import lax
from jax.experimental import pallas as pl
from jax.experimental.pallas import tpu as pltpu
from jax.experimental.pallas import tpu_sc as plsc

N = 10000
F = 128
NB = 80
NSLOT = 4
TILES = 16
CHUNKS_PER_CORE = 4
ROWS_PER_TILE = 624


def _hop_body(hsrc, epk, out, acc, gbuf, egb, rowb, idxb,
              sem_e, sem_g, sem_s):
    c = lax.axis_index("c")
    s = lax.axis_index("s")
    nbatch = epk.shape[0] // (TILES * 3 * NB)
    ngroup = nbatch // NSLOT
    r0 = s * ROWS_PER_TILE
    tail = N - TILES * ROWS_PER_TILE

    def chunk_body(k, _unused):
        g = c * CHUNKS_PER_CORE + k
        base = g * N

        def zrow(i, _):
            for j in range(8):
                gbuf[0, i, pl.ds(j * 16, 16)] = jnp.zeros((16,), jnp.float32)
            return 0
        lax.fori_loop(0, NB, zrow, 0)
        for t in range(ROWS_PER_TILE // NB):
            pltpu.sync_copy(gbuf.at[0], acc.at[pl.ds(r0 + t * NB, NB)])
        rem = ROWS_PER_TILE % NB
        if rem:
            pltpu.sync_copy(gbuf.at[0, pl.ds(0, rem)],
                            acc.at[pl.ds(r0 + (ROWS_PER_TILE // NB) * NB, rem)])

        @pl.when(s == TILES - 1)
        def _():
            pltpu.sync_copy(gbuf.at[0, pl.ds(0, tail)],
                            acc.at[pl.ds(TILES * ROWS_PER_TILE, tail)])
        plsc.subcore_barrier()

        bslot0 = s * nbatch

        def E(m, q):
            pltpu.async_copy(epk.at[pl.ds((bslot0 + m) * 3 * NB, 3 * NB)],
                             egb.at[pl.ds(q * 3 * NB, 3 * NB)], sem_e.at[q])

        def WE(q):
            pltpu.make_async_copy(epk.at[pl.ds(0, 3 * NB)],
                                  egb.at[pl.ds(q * 3 * NB, 3 * NB)],
                                  sem_e.at[q]).wait()

        def X(q):
            e0 = q * 3 * NB
            for j in range(NB // 16):
                idxb[q, pl.ds(j * 16, 16)] = egb[pl.ds(e0 + j * 16, 16)] + base
            for j in range(NB // 16):
                rowb[q, pl.ds(j * 16, 16)] = egb[pl.ds(e0 + 2 * NB + j * 16, 16)]

        def G(q):
            pltpu.async_copy(hsrc.at[idxb.at[q]], gbuf.at[q], sem_g.at[q])

        def WG(q):
            pltpu.make_async_copy(hsrc.at[pl.ds(0, NB)], gbuf.at[q], sem_g.at[q]).wait()

        def S(q):
            def scale(eb, _):
                vv = egb[pl.ds(q * 3 * NB + NB + eb * 16, 16)].astype(jnp.float32) * (2.0 ** -30)
                for e2 in range(16):
                    v = vv[e2]
                    e = eb * 16 + e2
                    for j in range(8):
                        gbuf[q, e, pl.ds(j * 16, 16)] = gbuf[q, e, pl.ds(j * 16, 16)] * v
                return 0
            lax.fori_loop(0, NB // 16, scale, 0)

        def C(q):
            pltpu.async_copy(gbuf.at[q], acc.at[rowb.at[q]], sem_s.at[q], add=True)

        def WS(q):
            pltpu.make_async_copy(hsrc.at[pl.ds(0, NB)], gbuf.at[q], sem_s.at[q]).wait()

        E(0, 0)
        E(1, 1)
        E(2, 2)
        WE(0)
        X(0)
        G(0)
        WE(1)
        X(1)
        G(1)

        def group(gi, _):
            m0 = gi * NSLOT
            for off in range(NSLOT):
                m = m0 + off

                @pl.when(m >= 2)
                def _():
                    WS((off + 2) % NSLOT)

                @pl.when(m + 3 <= nbatch - 1)
                def _():
                    E(m + 3, (off + 3) % NSLOT)

                @pl.when(m + 2 <= nbatch - 1)
                def _():
                    WE((off + 2) % NSLOT)
                    X((off + 2) % NSLOT)
                    G((off + 2) % NSLOT)
                WG(off)
                S(off)
                C(off)
            return 0
        lax.fori_loop(0, ngroup, group, 0)

        WS(2)
        WS(3)

        plsc.subcore_barrier()

        pltpu.sync_copy(acc.at[pl.ds(r0, ROWS_PER_TILE)],
                        out.at[pl.ds(g * N + r0, ROWS_PER_TILE)])

        @pl.when(s == TILES - 1)
        def _():
            pltpu.sync_copy(acc.at[pl.ds(TILES * ROWS_PER_TILE, tail)],
                            out.at[pl.ds(g * N + TILES * ROWS_PER_TILE, tail)])
        plsc.subcore_barrier()
        return 0

    lax.fori_loop(0, CHUNKS_PER_CORE, chunk_body, 0)


def _linear_body(h_ref, w_ref, b_ref, o_ref):
    o_ref[...] = jnp.dot(h_ref[...], w_ref[...],
                         preferred_element_type=jnp.float32) + b_ref[...]


def kernel(x, edge_row, edge_col, edge_val, W, b):
    B_, N_, F_ = x.shape
    E = edge_row.shape[0]

    unit = TILES * NB * NSLOT
    EP = ((E + unit - 1) // unit) * unit
    pad = EP - E
    ar = (jnp.arange(pad, dtype=jnp.int32) % N_)
    rowp = jnp.concatenate([edge_row, ar])
    colp = jnp.concatenate([edge_col, ar])
    valp = jnp.concatenate([edge_val, jnp.zeros((pad,), jnp.float32)])
    vbits = jnp.round(valp * (2.0 ** 30)).astype(jnp.int32)
    epk = jnp.stack([colp.reshape(EP // NB, NB),
                     vbits.reshape(EP // NB, NB),
                     rowp.reshape(EP // NB, NB)], axis=1).reshape(-1)

    mesh = plsc.VectorSubcoreMesh(core_axis_name="c", subcore_axis_name="s")
    hop = pl.kernel(
        _hop_body,
        mesh=mesh,
        out_type=jax.ShapeDtypeStruct((B_ * N_, F_), jnp.float32),
        scratch_types=[
            pltpu.VMEM_SHARED((N_, F_), jnp.float32),
            pltpu.VMEM((NSLOT, NB, F_), jnp.float32),
            pltpu.VMEM((NSLOT * 3 * NB,), jnp.int32),
            pltpu.VMEM((NSLOT, NB), jnp.int32),
            pltpu.VMEM((NSLOT, NB), jnp.int32),
            pltpu.SemaphoreType.DMA((NSLOT,)),
            pltpu.SemaphoreType.DMA((NSLOT,)),
            pltpu.SemaphoreType.DMA((NSLOT,)),
        ],
    )

    h = x.reshape(B_ * N_, F_)
    h = hop(h, epk)
    h = hop(h, epk)

    M = B_ * N_
    BLK = 2000
    out = pl.pallas_call(
        _linear_body,
        grid=(M // BLK,),
        in_specs=[
            pl.BlockSpec((BLK, F_), lambda i: (i, 0)),
            pl.BlockSpec((F_, W.shape[1]), lambda i: (0, 0)),
            pl.BlockSpec((W.shape[1],), lambda i: (0,)),
        ],
        out_specs=pl.BlockSpec((BLK, W.shape[1]), lambda i: (i, 0)),
        out_shape=jax.ShapeDtypeStruct((M, W.shape[1]), jnp.float32),
    )(h, W, b)
    return out.reshape(B_, N_, W.shape[1])
